# baseline (speedup 1.0000x reference)
import functools

import jax
import jax.numpy as jnp
from jax import lax
from jax.experimental import pallas as pl
from jax.experimental.pallas import tpu as pltpu
from jax.experimental.pallas import tpu_sc as plsc

N = 10000
E = 320000
D = 128
ALPHA = 0.2

NC = 2
NS = 16
NW = NC * NS
NPAD = 10240
CHUNK = 80
NCHUNK = 126
E_PER_TILE = NCHUNK * CHUNK
E_PAD = NW * E_PER_TILE
ROWS_STRIPE = NPAD // NS
NBUF = 3


def _mm_body(x_ref, w_ref, al_ref, ar_ref, wh_ref, el_ref, er_ref):
    wh = jnp.dot(x_ref[...], w_ref[...], preferred_element_type=jnp.float32)
    wh_ref[...] = wh
    el_ref[...] = jnp.sum(wh * al_ref[...], axis=1, keepdims=True)
    er_ref[...] = jnp.sum(wh * ar_ref[...], axis=1, keepdims=True)


def _matmul_part(x, W, a_left, a_right):
    BLK = 400
    grid = N // BLK
    return pl.pallas_call(
        _mm_body,
        grid=(grid,),
        in_specs=[
            pl.BlockSpec((BLK, D), lambda i: (i, 0)),
            pl.BlockSpec((D, D), lambda i: (0, 0)),
            pl.BlockSpec((1, D), lambda i: (0, 0)),
            pl.BlockSpec((1, D), lambda i: (0, 0)),
        ],
        out_specs=[
            pl.BlockSpec((BLK, D), lambda i: (i, 0)),
            pl.BlockSpec((BLK, 1), lambda i: (i, 0)),
            pl.BlockSpec((BLK, 1), lambda i: (i, 0)),
        ],
        out_shape=[
            jax.ShapeDtypeStruct((N, D), jnp.float32),
            jax.ShapeDtypeStruct((N, 1), jnp.float32),
            jax.ShapeDtypeStruct((N, 1), jnp.float32),
        ],
    )(x, W, a_left.reshape(1, D), a_right.reshape(1, D))


def _sc_edge_kernel(src3d, dst3d, el, er, wh):
    mesh = plsc.VectorSubcoreMesh(core_axis_name="c", subcore_axis_name="s")

    @functools.partial(
        pl.kernel,
        out_type=[
            jax.ShapeDtypeStruct((NC, NPAD, D), jnp.float32),
            jax.ShapeDtypeStruct((NW, NPAD), jnp.float32),
        ],
        mesh=mesh,
        compiler_params=pltpu.CompilerParams(needs_layout_passes=False),
        scratch_types=(
            [pltpu.VMEM((CHUNK, D), jnp.float32)] * NBUF
            + [pltpu.VMEM((CHUNK,), jnp.int32)] * NBUF
            + [pltpu.VMEM((CHUNK,), jnp.int32)] * NBUF
            + [pltpu.VMEM((CHUNK,), jnp.int32)] * NBUF
            + [pltpu.VMEM((CHUNK,), jnp.float32)] * NBUF
            + [pltpu.VMEM((CHUNK,), jnp.float32)] * NBUF
            + [pltpu.VMEM((NPAD,), jnp.float32)]
            + [pltpu.VMEM_SHARED((NPAD, D), jnp.float32)]
            + [pltpu.SemaphoreType.DMA] * (2 * NBUF)
        ),
    )
    def k(src_hbm, dst_hbm, el_hbm, er_hbm, wh_hbm, s_out, sum_out, *scr):
        rows = scr[0:NBUF]
        si = scr[NBUF:2 * NBUF]
        di = scr[2 * NBUF:3 * NBUF]
        ci = scr[3 * NBUF:4 * NBUF]
        elb = scr[4 * NBUF:5 * NBUF]
        erb = scr[5 * NBUF:6 * NBUF]
        esum_v = scr[6 * NBUF]
        s_sh = scr[6 * NBUF + 1]
        sems = scr[6 * NBUF + 2:7 * NBUF + 2]
        isems = scr[7 * NBUF + 2:8 * NBUF + 2]
        cid = lax.axis_index("c")
        sid = lax.axis_index("s")
        wid = cid * NS + sid
        zeros16 = jnp.zeros((16,), jnp.float32)
        rows0 = rows[0]

        def zero_rows(i, _):
            for j in range(D // 16):
                rows0[i, pl.ds(j * 16, 16)] = zeros16
            return 0
        lax.fori_loop(0, CHUNK, zero_rows, 0)

        def zero_esum(i, _):
            esum_v[pl.ds(i * 16, 16)] = zeros16
            return 0
        lax.fori_loop(0, NPAD // 16, zero_esum, 0)

        stripe0 = sid * ROWS_STRIPE
        for q in range(ROWS_STRIPE // CHUNK):
            pltpu.sync_copy(rows0, s_sh.at[pl.ds(stripe0 + q * CHUNK, CHUNK)])
        plsc.subcore_barrier()

        def fetch_idx(c, b):
            pltpu.async_copy(src_hbm.at[wid, c], si[b], isems[b])
            pltpu.async_copy(dst_hbm.at[wid, c], di[b], isems[b])

        def drain_idx(b):
            pltpu.make_async_copy(src_hbm.at[wid, 0], si[b], isems[b]).wait()
            pltpu.make_async_copy(src_hbm.at[wid, 0], di[b], isems[b]).wait()

        def issue_gathers(b):
            pltpu.async_copy(wh_hbm.at[di[b]], rows[b], sems[b])
            pltpu.async_copy(el_hbm.at[si[b]], elb[b], sems[b])
            pltpu.async_copy(er_hbm.at[di[b]], erb[b], sems[b])

        def drain_gathers(b):
            pltpu.make_async_copy(
                wh_hbm.at[pl.ds(0, CHUNK)], rows[b], sems[b]).wait()
            pltpu.make_async_copy(
                el_hbm.at[pl.ds(0, CHUNK)], elb[b], sems[b]).wait()
            pltpu.make_async_copy(
                el_hbm.at[pl.ds(0, CHUNK)], erb[b], sems[b]).wait()

        for b in range(NBUF):
            fetch_idx(b, b)
        for b in range(NBUF):
            drain_idx(b)
            issue_gathers(b)

        @pl.loop(0, NCHUNK, step=NBUF)
        def chunk_group(g):
            for b in range(NBUF):
                c = g + b
                rows_b, ci_b, el_b, er_b = rows[b], ci[b], elb[b], erb[b]
                si_b = si[b]
                drain_gathers(b)

                for k16 in range(CHUNK // 16):
                    ci_b[pl.ds(k16 * 16, 16)] = si_b[pl.ds(k16 * 16, 16)]

                @pl.when(c + NBUF < NCHUNK)
                def _():
                    fetch_idx(c + NBUF, b)

                for gi in range(CHUNK // 16):
                    ev = el_b[pl.ds(gi * 16, 16)] + er_b[pl.ds(gi * 16, 16)]
                    ev = jnp.where(ev > 0, ev, ALPHA * ev)
                    pvv = jnp.exp(ev)
                    s16 = ci_b[pl.ds(gi * 16, 16)]
                    plsc.addupdate_scatter(esum_v, [s16], pvv)
                    for i in range(16):
                        r = gi * 16 + i
                        pr = pvv[i]
                        for j in range(D // 16):
                            rows_b[r, pl.ds(j * 16, 16)] = (
                                rows_b[r, pl.ds(j * 16, 16)] * pr)

                pltpu.sync_copy(rows_b, s_sh.at[ci_b], add=True)

                @pl.when(c + NBUF < NCHUNK)
                def _():
                    drain_idx(b)
                    issue_gathers(b)

        plsc.subcore_barrier()

        pltpu.sync_copy(esum_v, sum_out.at[wid])
        for q in range(ROWS_STRIPE // CHUNK):
            base = stripe0 + q * CHUNK
            pltpu.sync_copy(s_sh.at[pl.ds(base, CHUNK)], rows0)
            pltpu.sync_copy(rows0, s_out.at[cid, pl.ds(base, CHUNK)])

    return k(src3d, dst3d, el, er, wh)


def _combine_body(s_ref, sum_ref, o_ref):
    s = s_ref[0] + s_ref[1]
    d = jnp.sum(sum_ref[...], axis=0) + 1e-9
    o_ref[...] = jnp.maximum(s * (1.0 / d)[:, None], 0.0)


def _combine(s_parts, sum_parts):
    BLK = 512
    grid = NPAD // BLK
    return pl.pallas_call(
        _combine_body,
        grid=(grid,),
        in_specs=[
            pl.BlockSpec((NC, BLK, D), lambda i: (0, i, 0)),
            pl.BlockSpec((NW, BLK), lambda i: (0, i)),
        ],
        out_specs=pl.BlockSpec((BLK, D), lambda i: (i, 0)),
        out_shape=jax.ShapeDtypeStruct((NPAD, D), jnp.float32),
    )(s_parts, sum_parts)


def kernel(x, edge_index, W, a_left, a_right):
    wh, el2, er2 = _matmul_part(x, W, a_left, a_right)
    el = jnp.pad(el2.reshape(N), (0, NPAD - N))
    er = jnp.pad(er2.reshape(N), (0, NPAD - N))
    src = jnp.concatenate(
        [edge_index[0], jnp.full((E_PAD - E,), N, jnp.int32)])
    dst = jnp.concatenate(
        [edge_index[1], jnp.zeros((E_PAD - E,), jnp.int32)])
    src3d = src.reshape(NW, NCHUNK, CHUNK)
    dst3d = dst.reshape(NW, NCHUNK, CHUNK)
    s_parts, sum_parts = _sc_edge_kernel(src3d, dst3d, el, er, wh)
    out = _combine(s_parts, sum_parts)
    return out[:N]


# speedup vs baseline: 1.0669x; 1.0669x over previous
import functools

import jax
import jax.numpy as jnp
from jax import lax
from jax.experimental import pallas as pl
from jax.experimental.pallas import tpu as pltpu
from jax.experimental.pallas import tpu_sc as plsc

N = 10000
E = 320000
D = 128
ALPHA = 0.2

NC = 2
NS = 16
NW = NC * NS
NPAD = 10240
CHUNK = 80
NCHUNK = 126
E_PER_TILE = NCHUNK * CHUNK
E_PAD = NW * E_PER_TILE
ROWS_STRIPE = NPAD // NS
NBUF = 2


def _mm_body(x_ref, w_ref, al_ref, ar_ref, wh_ref, el_ref, er_ref):
    wh = jnp.dot(x_ref[...], w_ref[...], preferred_element_type=jnp.float32)
    wh_ref[...] = wh
    el_ref[...] = jnp.sum(wh * al_ref[...], axis=1, keepdims=True)
    er_ref[...] = jnp.sum(wh * ar_ref[...], axis=1, keepdims=True)


def _matmul_part(x, W, a_left, a_right):
    BLK = 400
    grid = N // BLK
    return pl.pallas_call(
        _mm_body,
        grid=(grid,),
        in_specs=[
            pl.BlockSpec((BLK, D), lambda i: (i, 0)),
            pl.BlockSpec((D, D), lambda i: (0, 0)),
            pl.BlockSpec((1, D), lambda i: (0, 0)),
            pl.BlockSpec((1, D), lambda i: (0, 0)),
        ],
        out_specs=[
            pl.BlockSpec((BLK, D), lambda i: (i, 0)),
            pl.BlockSpec((BLK, 1), lambda i: (i, 0)),
            pl.BlockSpec((BLK, 1), lambda i: (i, 0)),
        ],
        out_shape=[
            jax.ShapeDtypeStruct((N, D), jnp.float32),
            jax.ShapeDtypeStruct((N, 1), jnp.float32),
            jax.ShapeDtypeStruct((N, 1), jnp.float32),
        ],
    )(x, W, a_left.reshape(1, D), a_right.reshape(1, D))


def _sc_edge_kernel(src3d, dst3d, el, er, wh):
    mesh = plsc.VectorSubcoreMesh(core_axis_name="c", subcore_axis_name="s")

    @functools.partial(
        pl.kernel,
        out_type=[
            jax.ShapeDtypeStruct((NC, NPAD, D), jnp.float32),
            jax.ShapeDtypeStruct((NC, NPAD), jnp.float32),
        ],
        mesh=mesh,
        compiler_params=pltpu.CompilerParams(needs_layout_passes=False),
        scratch_types=(
            [pltpu.VMEM((CHUNK, D), jnp.float32)] * NBUF
            + [pltpu.VMEM((E_PER_TILE,), jnp.int32)]
            + [pltpu.VMEM((E_PER_TILE,), jnp.int32)]
            + [pltpu.VMEM((CHUNK,), jnp.int32)]
            + [pltpu.VMEM((CHUNK,), jnp.float32)] * NBUF
            + [pltpu.VMEM((CHUNK,), jnp.float32)] * NBUF
            + [pltpu.VMEM((CHUNK,), jnp.float32)]
            + [pltpu.VMEM((ROWS_STRIPE,), jnp.float32)]
            + [pltpu.VMEM_SHARED((NPAD, D), jnp.float32)]
            + [pltpu.VMEM_SHARED((NPAD,), jnp.float32)]
            + [pltpu.SemaphoreType.DMA] * NBUF
        ),
    )
    def k(src_hbm, dst_hbm, el_hbm, er_hbm, wh_hbm, s_out, sum_out, *scr):
        rows = scr[0:NBUF]
        stbl = scr[NBUF]
        dtbl = scr[NBUF + 1]
        ci = scr[NBUF + 2]
        elb = scr[NBUF + 3:2 * NBUF + 3]
        erb = scr[2 * NBUF + 3:3 * NBUF + 3]
        pv = scr[3 * NBUF + 3]
        zb = scr[3 * NBUF + 4]
        s_sh = scr[3 * NBUF + 5]
        esum_sh = scr[3 * NBUF + 6]
        sems = scr[3 * NBUF + 7:4 * NBUF + 7]
        cid = lax.axis_index("c")
        sid = lax.axis_index("s")
        wid = cid * NS + sid
        zeros16 = jnp.zeros((16,), jnp.float32)
        rows0 = rows[0]

        def zero_rows(i, _):
            for j in range(D // 16):
                rows0[i, pl.ds(j * 16, 16)] = zeros16
            return 0
        lax.fori_loop(0, CHUNK, zero_rows, 0)

        def zero_zb(i, _):
            zb[pl.ds(i * 16, 16)] = zeros16
            return 0
        lax.fori_loop(0, ROWS_STRIPE // 16, zero_zb, 0)

        stripe0 = sid * ROWS_STRIPE
        for q in range(ROWS_STRIPE // CHUNK):
            pltpu.sync_copy(rows0, s_sh.at[pl.ds(stripe0 + q * CHUNK, CHUNK)])
        pltpu.sync_copy(zb, esum_sh.at[pl.ds(stripe0, ROWS_STRIPE)])

        pltpu.sync_copy(src_hbm.at[wid], stbl)
        pltpu.sync_copy(dst_hbm.at[wid], dtbl)
        plsc.subcore_barrier()

        def issue_gathers(c, b):
            pltpu.async_copy(
                wh_hbm.at[dtbl.at[pl.ds(c * CHUNK, CHUNK)]], rows[b], sems[b])
            pltpu.async_copy(
                el_hbm.at[stbl.at[pl.ds(c * CHUNK, CHUNK)]], elb[b], sems[b])
            pltpu.async_copy(
                er_hbm.at[dtbl.at[pl.ds(c * CHUNK, CHUNK)]], erb[b], sems[b])

        def drain_gathers(b):
            pltpu.make_async_copy(
                wh_hbm.at[pl.ds(0, CHUNK)], rows[b], sems[b]).wait()
            pltpu.make_async_copy(
                el_hbm.at[pl.ds(0, CHUNK)], elb[b], sems[b]).wait()
            pltpu.make_async_copy(
                el_hbm.at[pl.ds(0, CHUNK)], erb[b], sems[b]).wait()

        for b in range(NBUF):
            issue_gathers(b, b)

        @pl.loop(0, NCHUNK, step=NBUF)
        def chunk_group(g):
            for b in range(NBUF):
                c = g + b
                rows_b, el_b, er_b = rows[b], elb[b], erb[b]
                drain_gathers(b)

                def ci_copy(k16, _):
                    ci[pl.ds(k16 * 16, 16)] = stbl[
                        pl.ds(c * CHUNK + k16 * 16, 16)]
                    return 0
                lax.fori_loop(0, CHUNK // 16, ci_copy, 0)

                for gi in range(CHUNK // 16):
                    ev = el_b[pl.ds(gi * 16, 16)] + er_b[pl.ds(gi * 16, 16)]
                    ev = jnp.where(ev > 0, ev, ALPHA * ev)
                    pvv = jnp.exp(ev)
                    pv[pl.ds(gi * 16, 16)] = pvv
                    for i in range(16):
                        r = gi * 16 + i
                        pr = pvv[i]
                        for j in range(D // 16):
                            rows_b[r, pl.ds(j * 16, 16)] = (
                                rows_b[r, pl.ds(j * 16, 16)] * pr)

                pltpu.sync_copy(pv, esum_sh.at[ci], add=True)
                pltpu.sync_copy(rows_b, s_sh.at[ci], add=True)

                @pl.when(c + NBUF < NCHUNK)
                def _():
                    issue_gathers(c + NBUF, b)

        plsc.subcore_barrier()

        pltpu.sync_copy(esum_sh.at[pl.ds(stripe0, ROWS_STRIPE)], zb)
        pltpu.sync_copy(zb, sum_out.at[cid, pl.ds(stripe0, ROWS_STRIPE)])
        for q in range(ROWS_STRIPE // CHUNK):
            base = stripe0 + q * CHUNK
            pltpu.sync_copy(s_sh.at[pl.ds(base, CHUNK)], rows0)
            pltpu.sync_copy(rows0, s_out.at[cid, pl.ds(base, CHUNK)])

    return k(src3d, dst3d, el, er, wh)


def _combine_body(s_ref, sum_ref, o_ref):
    s = s_ref[0] + s_ref[1]
    d = sum_ref[0] + sum_ref[1] + 1e-9
    o_ref[...] = jnp.maximum(s * (1.0 / d)[:, None], 0.0)


def _combine(s_parts, sum_parts):
    BLK = 512
    grid = NPAD // BLK
    return pl.pallas_call(
        _combine_body,
        grid=(grid,),
        in_specs=[
            pl.BlockSpec((NC, BLK, D), lambda i: (0, i, 0)),
            pl.BlockSpec((NC, BLK), lambda i: (0, i)),
        ],
        out_specs=pl.BlockSpec((BLK, D), lambda i: (i, 0)),
        out_shape=jax.ShapeDtypeStruct((NPAD, D), jnp.float32),
    )(s_parts, sum_parts)


def kernel(x, edge_index, W, a_left, a_right):
    wh, el2, er2 = _matmul_part(x, W, a_left, a_right)
    el = jnp.pad(el2.reshape(N), (0, NPAD - N))
    er = jnp.pad(er2.reshape(N), (0, NPAD - N))
    src = jnp.concatenate(
        [edge_index[0], jnp.full((E_PAD - E,), N, jnp.int32)])
    dst = jnp.concatenate(
        [edge_index[1], jnp.zeros((E_PAD - E,), jnp.int32)])
    src2d = src.reshape(NW, E_PER_TILE)
    dst2d = dst.reshape(NW, E_PER_TILE)
    s_parts, sum_parts = _sc_edge_kernel(src2d, dst2d, el, er, wh)
    out = _combine(s_parts, sum_parts)
    return out[:N]
